# baseline (speedup 1.0000x reference)
import jax
import jax.numpy as jnp
from jax import lax
from jax.experimental import pallas as pl
from jax.experimental.pallas import tpu as pltpu, tpu_sc as plsc

N = 10000
E = 320000
D = 128
HALF = 64
NC = 2
NS = 16
L = 16
NPAD = 10240
NPT = NPAD // NS
B = 128
NB = 160
EPT = NB * B
EPAD = EPT * NS
RMASK = (1 << 14) - 1


def _rsqrt16(v):
    s = 0.5 * (1.0 + v)
    for _ in range(12):
        s = 0.5 * (s + v / s)
    return 1.0 / s


def _body(x2d, rch, ewh, out,
          rc2d, nrm1d, dis_t, irow, icol, buf0, buf1, buf2, buf3, dchunk,
          acc, deg_s,
          sem, gsem0, gsem1, gsem2, gsem3, ssem0, ssem1, ssem2, ssem3):
    c = lax.axis_index("c")
    s = lax.axis_index("s")
    nbase = s * NPT
    bufs = (buf0, buf1, buf2, buf3)
    gsems = (gsem0, gsem1, gsem2, gsem3)
    ssems = (ssem0, ssem1, ssem2, ssem3)
    iota16 = lax.iota(jnp.int32, L)

    with jax.named_scope("stage"):
        pltpu.sync_copy(rch.at[s], rc2d)
        pltpu.sync_copy(ewh.at[s], nrm1d)

    with jax.named_scope("degphase"):
        @pl.loop(0, NPT // L)
        def _(i):
            dchunk[pl.ds(i * L, L)] = jnp.full((L,), 1.0, jnp.float32)
        pltpu.sync_copy(dchunk, deg_s.at[pl.ds(nbase, NPT)])
        plsc.subcore_barrier()

        @pl.loop(0, NB, step=8)
        def _(p):
            for u in range(8):
                for g in range(B // L):
                    sl = pl.ds(g * L, L)
                    irow[u, sl] = rc2d[p + u, sl] & RMASK
            for u in range(8):
                eoff = pl.multiple_of((p + u) * B, B)
                pltpu.make_async_copy(
                    nrm1d.at[pl.ds(eoff, B)], deg_s.at[irow.at[u]], sem
                ).start(add=True)
            for u in range(8):
                eoff = pl.multiple_of((p + u) * B, B)
                pltpu.make_async_copy(
                    nrm1d.at[pl.ds(eoff, B)], deg_s.at[irow.at[u]], sem).wait()
        plsc.subcore_barrier()

        pltpu.sync_copy(deg_s.at[pl.ds(nbase, NPT)], dchunk)

        @pl.loop(0, NPT // L)
        def _(i):
            dchunk[pl.ds(i * L, L)] = _rsqrt16(dchunk[pl.ds(i * L, L)])
        pltpu.sync_copy(dchunk, deg_s.at[pl.ds(nbase, NPT)])
        plsc.subcore_barrier()
        pltpu.sync_copy(deg_s, dis_t)

    with jax.named_scope("normphase"):
        @pl.loop(0, NB)
        def _(b):
            @pl.loop(0, B // L)
            def _(g):
                sl = pl.ds(g * L, L)
                fl = pl.ds(b * B + g * L, L)
                rc = rc2d[b, sl]
                ir = rc & RMASK
                ic = lax.shift_right_logical(rc, 14)
                ew = nrm1d[fl]
                nrm1d[fl] = plsc.load_gather(dis_t, [ir]) * ew \
                    * plsc.load_gather(dis_t, [ic])

    def _scale_buf_by_dis2(bu, rb):
        @pl.loop(0, B, init_carry=jnp.full((L,), rb, jnp.int32), unroll=4)
        def _(i, iv):
            dv = plsc.load_gather(dis_t, [iv])
            d2 = dv * dv
            for j in range(HALF // L):
                bu[i, pl.ds(j * L, L)] = bu[i, pl.ds(j * L, L)] * d2
            return iv + 1

    def _scatter_out(rb, hoff, slot):
        for g in range(B // L):
            n16 = iota16 + (rb + g * L)
            irow[slot, pl.ds(g * L, L)] = jnp.where(
                n16 < N, n16 * 6 + hoff, 6 * N)
        pltpu.sync_copy(buf0, out.at[irow.at[slot]])

    def _init_acc():
        for k in range(NPT // B):
            rb = nbase + k * B
            for g in range(B // L):
                n16 = iota16 + (rb + g * L)
                irow[0, pl.ds(g * L, L)] = jnp.where(
                    n16 < N, n16 * 2 + c, 0)
            pltpu.sync_copy(x2d.at[irow.at[0]], buf0)
            _scatter_out(rb, c, 1)
            _scale_buf_by_dis2(buf0, rb)
            pltpu.sync_copy(buf0, acc.at[pl.ds(rb, B)])

    def _fill_ring(b, roff):
        u8 = b % 8
        for g in range(B // L):
            sl = pl.ds(g * L, L)
            rc = rc2d[b, sl]
            irow[u8, sl] = (rc & RMASK) * 6 + roff
            icol[u8, sl] = lax.shift_right_logical(rc, 14)

    def _propagate(roff):
        def gather(action, b, u):
            d = pltpu.make_async_copy(
                out.at[irow.at[b % 8]], bufs[u], gsems[u])
            d.start() if action == "start" else d.wait()

        def scatter(action, b, u):
            d = pltpu.make_async_copy(
                bufs[u], acc.at[icol.at[b % 8]], ssems[u])
            d.start(add=True) if action == "start" else d.wait()

        _fill_ring(0, roff)
        _fill_ring(1, roff)
        gather("start", 0, 0)
        gather("start", 1, 1)

        @pl.loop(0, NB, step=4)
        def _(pb):
            for u in range(4):
                b = pb + u

                @pl.when(b >= 2)
                def _():
                    scatter("wait", b - 2, (u + 2) % 4)

                @pl.when(b + 2 < NB)
                def _():
                    _fill_ring(b + 2, roff)
                    gather("start", b + 2, (u + 2) % 4)
                gather("wait", b, u)

                bu = bufs[u]

                @pl.loop(0, B, init_carry=jnp.full((L,), b * B, jnp.int32),
                         unroll=4)
                def _(i, iv):
                    nv = plsc.load_gather(nrm1d, [iv])
                    for j in range(HALF // L):
                        bu[i, pl.ds(j * L, L)] = bu[i, pl.ds(j * L, L)] * nv
                    return iv + 1
                scatter("start", b, u)
        scatter("wait", NB - 2, 2)
        scatter("wait", NB - 1, 3)

    def _writeout(hoff, reinit):
        for k in range(NPT // B):
            rb = nbase + k * B
            pltpu.sync_copy(acc.at[pl.ds(rb, B)], buf0)
            _scatter_out(rb, hoff, 0)
            if reinit:
                _scale_buf_by_dis2(buf0, rb)
                pltpu.sync_copy(buf0, acc.at[pl.ds(rb, B)])

    with jax.named_scope("init_acc"):
        _init_acc()
    plsc.subcore_barrier()
    with jax.named_scope("prop1"):
        _propagate(c)
    plsc.subcore_barrier()
    with jax.named_scope("writeout1"):
        _writeout(2 + c, reinit=True)
    plsc.subcore_barrier()
    with jax.named_scope("prop2"):
        _propagate(2 + c)
    plsc.subcore_barrier()
    with jax.named_scope("writeout2"):
        _writeout(4 + c, reinit=False)


def _run(x2d, rch, ewh):
    mesh = plsc.VectorSubcoreMesh(core_axis_name="c", subcore_axis_name="s")
    f = pl.kernel(
        _body,
        out_type=jax.ShapeDtypeStruct((6 * N + B, HALF), jnp.float32),
        mesh=mesh,
        compiler_params=pltpu.CompilerParams(
            needs_layout_passes=False, use_tc_tiling_on_sc=False,
            disable_bounds_checks=True),
        scratch_types=[
            pltpu.VMEM((NB, B), jnp.int32),
            pltpu.VMEM((NB * B,), jnp.float32),
            pltpu.VMEM((NPAD,), jnp.float32),
            pltpu.VMEM((8, B), jnp.int32),
            pltpu.VMEM((8, B), jnp.int32),
            pltpu.VMEM((B, HALF), jnp.float32),
            pltpu.VMEM((B, HALF), jnp.float32),
            pltpu.VMEM((B, HALF), jnp.float32),
            pltpu.VMEM((B, HALF), jnp.float32),
            pltpu.VMEM((NPT,), jnp.float32),
            pltpu.VMEM_SHARED((NPAD, HALF), jnp.float32),
            pltpu.VMEM_SHARED((NPAD,), jnp.float32),
        ] + [pltpu.SemaphoreType.DMA] * 9,
    )
    return f(x2d, rch, ewh)


def kernel(x, edge_index, edge_weight, has_feature):
    x = x.astype(jnp.float32)
    row = edge_index[0]
    col = edge_index[1]
    pad = EPAD - E
    rc = row.astype(jnp.uint32) | (col.astype(jnp.uint32) << 14)
    pad_ids = jnp.arange(pad, dtype=jnp.uint32) % jnp.uint32(N)
    rc_pad = pad_ids | (pad_ids << 14)
    rch = jnp.concatenate([rc, rc_pad]).astype(jnp.int32).reshape(NS, NB, B)
    ewh = jnp.concatenate(
        [edge_weight, jnp.zeros((pad,), edge_weight.dtype)]).reshape(NS, NB * B)
    x2d = x.reshape(2 * N, HALF)
    out = _run(x2d, rch, ewh)
    return out[:6 * N].reshape(N, 6 * HALF)


# speedup vs baseline: 1.1431x; 1.1431x over previous
import jax
import jax.numpy as jnp
from jax import lax
from jax.experimental import pallas as pl
from jax.experimental.pallas import tpu as pltpu, tpu_sc as plsc

N = 10000
E = 320000
D = 128
HALF = 64
NC = 2
NS = 16
L = 16
NPAD = 10240
NPT = NPAD // NS
B = 128
NB = 160
EPT = NB * B
EPAD = EPT * NS
RMASK = (1 << 14) - 1


def _rsqrt16(v):
    s = 0.5 * (1.0 + v)
    for _ in range(12):
        s = 0.5 * (s + v / s)
    return 1.0 / s


def _body(x2d, rch, ewh, out,
          rc2d, nrm1d, dis_t, irow, icol, i16, buf0, buf1, buf2, buf3, dchunk,
          acc, deg_s,
          sem, gsem0, gsem1, gsem2, gsem3, ssem0, ssem1, ssem2, ssem3):
    c = lax.axis_index("c")
    s = lax.axis_index("s")
    nbase = s * NPT
    bufs = (buf0, buf1, buf2, buf3)
    gsems = (gsem0, gsem1, gsem2, gsem3)
    ssems = (ssem0, ssem1, ssem2, ssem3)
    iota16 = lax.iota(jnp.int32, L)

    with jax.named_scope("stage"):
        pltpu.sync_copy(rch.at[s], rc2d)
        pltpu.sync_copy(ewh.at[s], nrm1d)

    with jax.named_scope("degphase"):
        @pl.loop(0, NPT // L)
        def _(i):
            dchunk[pl.ds(i * L, L)] = jnp.full((L,), 1.0, jnp.float32)
        pltpu.sync_copy(dchunk, deg_s.at[pl.ds(nbase, NPT)])
        plsc.subcore_barrier()

        @pl.loop(0, NB, step=8)
        def _(p):
            for u in range(8):
                for g in range(B // L):
                    sl = pl.ds(g * L, L)
                    irow[u, sl] = rc2d[p + u, sl] & RMASK
            for u in range(8):
                eoff = pl.multiple_of((p + u) * B, B)
                pltpu.make_async_copy(
                    nrm1d.at[pl.ds(eoff, B)], deg_s.at[irow.at[u]], sem
                ).start(add=True)
            for u in range(8):
                eoff = pl.multiple_of((p + u) * B, B)
                pltpu.make_async_copy(
                    nrm1d.at[pl.ds(eoff, B)], deg_s.at[irow.at[u]], sem).wait()
        plsc.subcore_barrier()

        pltpu.sync_copy(deg_s.at[pl.ds(nbase, NPT)], dchunk)

        @pl.loop(0, NPT // L)
        def _(i):
            dchunk[pl.ds(i * L, L)] = _rsqrt16(dchunk[pl.ds(i * L, L)])
        pltpu.sync_copy(dchunk, deg_s.at[pl.ds(nbase, NPT)])
        plsc.subcore_barrier()
        pltpu.sync_copy(deg_s, dis_t)

    with jax.named_scope("normphase"):
        @pl.loop(0, NB)
        def _(b):
            @pl.loop(0, B // L)
            def _(g):
                sl = pl.ds(g * L, L)
                fl = pl.ds(b * B + g * L, L)
                rc = rc2d[b, sl]
                ir = rc & RMASK
                ic = lax.shift_right_logical(rc, 14)
                ew = nrm1d[fl]
                nrm1d[fl] = plsc.load_gather(dis_t, [ir]) * ew \
                    * plsc.load_gather(dis_t, [ic])

    def _scale_buf_by_dis2(bu, rb):
        @pl.loop(0, B, init_carry=jnp.full((L,), rb, jnp.int32), unroll=4)
        def _(i, iv):
            dv = plsc.load_gather(dis_t, [iv])
            d2 = dv * dv
            for j in range(HALF // L):
                bu[i, pl.ds(j * L, L)] = bu[i, pl.ds(j * L, L)] * d2
            return iv + 1

    def _scatter_out(rb, hoff, slot):
        @pl.when(rb + B <= N)
        def _():
            for g in range(B // L):
                n16 = iota16 + (rb + g * L)
                irow[slot, pl.ds(g * L, L)] = n16 * 6 + hoff
            pltpu.sync_copy(buf0, out.at[irow.at[slot]])

        @pl.when((rb < N) & (rb + B > N))
        def _():
            i16[slot, :] = (iota16 + rb) * 6 + hoff
            pltpu.sync_copy(buf0.at[pl.ds(0, L)], out.at[i16.at[slot]])

    def _init_acc():
        for k in range(NPT // B):
            rb = nbase + k * B
            for g in range(B // L):
                n16 = iota16 + (rb + g * L)
                irow[0, pl.ds(g * L, L)] = jnp.where(
                    n16 < N, n16 * 2 + c, 0)
            pltpu.sync_copy(x2d.at[irow.at[0]], buf0)
            _scatter_out(rb, c, 1)
            _scale_buf_by_dis2(buf0, rb)
            pltpu.sync_copy(buf0, acc.at[pl.ds(rb, B)])

    def _fill_ring(b, roff):
        u8 = b % 8
        for g in range(B // L):
            sl = pl.ds(g * L, L)
            rc = rc2d[b, sl]
            irow[u8, sl] = (rc & RMASK) * 6 + roff
            icol[u8, sl] = lax.shift_right_logical(rc, 14)

    def _propagate(roff):
        def gather(action, b, u):
            d = pltpu.make_async_copy(
                out.at[irow.at[b % 8]], bufs[u], gsems[u])
            d.start() if action == "start" else d.wait()

        def scatter(action, b, u):
            d = pltpu.make_async_copy(
                bufs[u], acc.at[icol.at[b % 8]], ssems[u])
            d.start(add=True) if action == "start" else d.wait()

        _fill_ring(0, roff)
        _fill_ring(1, roff)
        gather("start", 0, 0)
        gather("start", 1, 1)

        @pl.loop(0, NB, step=4)
        def _(pb):
            for u in range(4):
                b = pb + u

                @pl.when(b >= 2)
                def _():
                    scatter("wait", b - 2, (u + 2) % 4)

                @pl.when(b + 2 < NB)
                def _():
                    _fill_ring(b + 2, roff)
                    gather("start", b + 2, (u + 2) % 4)
                gather("wait", b, u)

                bu = bufs[u]

                @pl.loop(0, B, init_carry=jnp.full((L,), b * B, jnp.int32),
                         unroll=4)
                def _(i, iv):
                    nv = plsc.load_gather(nrm1d, [iv])
                    for j in range(HALF // L):
                        bu[i, pl.ds(j * L, L)] = bu[i, pl.ds(j * L, L)] * nv
                    return iv + 1
                scatter("start", b, u)
        scatter("wait", NB - 2, 2)
        scatter("wait", NB - 1, 3)

    def _writeout(hoff, reinit):
        for k in range(NPT // B):
            rb = nbase + k * B
            pltpu.sync_copy(acc.at[pl.ds(rb, B)], buf0)
            _scatter_out(rb, hoff, 0)
            if reinit:
                _scale_buf_by_dis2(buf0, rb)
                pltpu.sync_copy(buf0, acc.at[pl.ds(rb, B)])

    with jax.named_scope("init_acc"):
        _init_acc()
    plsc.subcore_barrier()
    with jax.named_scope("prop1"):
        _propagate(c)
    plsc.subcore_barrier()
    with jax.named_scope("writeout1"):
        _writeout(2 + c, reinit=True)
    plsc.subcore_barrier()
    with jax.named_scope("prop2"):
        _propagate(2 + c)
    plsc.subcore_barrier()
    with jax.named_scope("writeout2"):
        _writeout(4 + c, reinit=False)


def _run(x2d, rch, ewh):
    mesh = plsc.VectorSubcoreMesh(core_axis_name="c", subcore_axis_name="s")
    f = pl.kernel(
        _body,
        out_type=jax.ShapeDtypeStruct((6 * N, HALF), jnp.float32),
        mesh=mesh,
        compiler_params=pltpu.CompilerParams(
            needs_layout_passes=False, use_tc_tiling_on_sc=False,
            disable_bounds_checks=True),
        scratch_types=[
            pltpu.VMEM((NB, B), jnp.int32),
            pltpu.VMEM((NB * B,), jnp.float32),
            pltpu.VMEM((NPAD,), jnp.float32),
            pltpu.VMEM((8, B), jnp.int32),
            pltpu.VMEM((8, B), jnp.int32),
            pltpu.VMEM((8, L), jnp.int32),
            pltpu.VMEM((B, HALF), jnp.float32),
            pltpu.VMEM((B, HALF), jnp.float32),
            pltpu.VMEM((B, HALF), jnp.float32),
            pltpu.VMEM((B, HALF), jnp.float32),
            pltpu.VMEM((NPT,), jnp.float32),
            pltpu.VMEM_SHARED((NPAD, HALF), jnp.float32),
            pltpu.VMEM_SHARED((NPAD,), jnp.float32),
        ] + [pltpu.SemaphoreType.DMA] * 9,
    )
    return f(x2d, rch, ewh)


def kernel(x, edge_index, edge_weight, has_feature):
    x = x.astype(jnp.float32)
    row = edge_index[0]
    col = edge_index[1]
    pad = EPAD - E
    rc = row.astype(jnp.uint32) | (col.astype(jnp.uint32) << 14)
    pad_ids = jnp.arange(pad, dtype=jnp.uint32) % jnp.uint32(N)
    rc_pad = pad_ids | (pad_ids << 14)
    rch = jnp.concatenate([rc, rc_pad]).astype(jnp.int32).reshape(NS, NB, B)
    ewh = jnp.concatenate(
        [edge_weight, jnp.zeros((pad,), edge_weight.dtype)]).reshape(NS, NB * B)
    x2d = x.reshape(2 * N, HALF)
    out = _run(x2d, rch, ewh)
    return out.reshape(N, 6 * HALF)
